# baseline (speedup 1.0000x reference)
import functools

import jax
import jax.numpy as jnp
from jax import lax
from jax.experimental import pallas as pl
from jax.experimental.pallas import tpu as pltpu
from jax.experimental.pallas import tpu_sc as plsc


def _build_body(a_ref, s_ref, o_ref, *, n, qn):
    j2 = pl.program_id(0)
    d = n * qn
    qa = lax.broadcasted_iota(jnp.int32, (qn, 2 * d), 0)
    bi = lax.broadcasted_iota(jnp.int32, (qn, 2 * d), 1)
    cb = bi % d
    qb = cb % qn
    kb = cb // qn
    half = bi // d
    eyem = (qa == qb).astype(jnp.float32)
    acc = None
    for i in range(n):
        coef = jnp.zeros((qn, 2 * d), jnp.float32)
        for k in range(n):
            msk = (kb == k)
            coef = coef + jnp.where(msk & (half == 0), a_ref[i, j2, k], 0.0)
            coef = coef + jnp.where(msk & (half == 1), a_ref[i, j2 + 2, k], 0.0)
        t = jnp.dot(s_ref[i], eyem * coef, preferred_element_type=jnp.float32)
        acc = t if acc is None else acc + t
    o_ref[...] = acc


def _build_weight(A, S, n, qn):
    mn = S.shape[1]
    d = n * qn
    pb = 5000
    npb = mn // pb
    return pl.pallas_call(
        functools.partial(_build_body, n=n, qn=qn),
        grid=(2, npb),
        in_specs=[
            pl.BlockSpec(memory_space=pltpu.SMEM),
            pl.BlockSpec((n, pb, qn), lambda j2, p: (0, p, 0)),
        ],
        out_specs=pl.BlockSpec((pb, 2 * d), lambda j2, p: (j2 * npb + p, 0)),
        out_shape=jax.ShapeDtypeStruct((2 * mn, 2 * d), jnp.float32),
    )(A, S)


def _gather_rows(table, idx, L, d):
    info = plsc.get_sparse_core_info()
    nw = info.num_cores * info.num_subcores
    t = idx.shape[0]
    per_w = t // nw
    ch = 128
    n_ch = per_w // ch
    ndt = d // 8
    mesh = plsc.VectorSubcoreMesh(core_axis_name="c", subcore_axis_name="s")

    @functools.partial(
        pl.kernel,
        mesh=mesh,
        out_type=jax.ShapeDtypeStruct((L, ndt, nw, 8, ch), jnp.float32),
        scratch_types=[
            pltpu.VMEM((per_w,), jnp.int32),
            pltpu.VMEM((ch, d), jnp.float32),
            pltpu.VMEM((ch, d), jnp.float32),
            pltpu.VMEM((ch, d), jnp.float32),
            pltpu.VMEM((ch, d), jnp.float32),
            pltpu.VMEM((ndt, 8, ch + 1), jnp.float32),
            pltpu.SemaphoreType.DMA,
            pltpu.SemaphoreType.DMA,
            pltpu.SemaphoreType.DMA,
            pltpu.SemaphoreType.DMA,
        ],
        compiler_params=pltpu.CompilerParams(
            use_tc_tiling_on_sc=False, needs_layout_passes=False),
    )
    def k(w_hbm, i_hbm, o_hbm, idx_v, rows0, rows1, rows2, rows3, stage,
          sem0, sem1, sem2, sem3):
        wid = lax.axis_index("s") * info.num_cores + lax.axis_index("c")
        base = wid * per_w
        pltpu.sync_copy(i_hbm.at[pl.ds(base, per_w)], idx_v)

        rows = (rows0, rows1, rows2, rows3)
        sems = (sem0, sem1, sem2, sem3)
        lane = lax.broadcasted_iota(jnp.int32, (16,), 0)
        dtv = [g * 2 + lane // 8 for g in range(d // 16)]
        dsv = lane % 8

        def start(c, b):
            return pltpu.async_copy(
                w_hbm.at[idx_v.at[pl.ds(c * ch, ch)]], rows[b], sems[b])

        def consume(c, b):
            ng = d // 16
            vs = [rows[b][0, pl.ds(g * 16, 16)] for g in range(ng)]
            for bl in range(ch):
                nxt = ([rows[b][bl + 1, pl.ds(g * 16, 16)] for g in range(ng)]
                       if bl + 1 < ch else None)
                blv = jnp.full((16,), bl, jnp.int32)
                for g in range(ng):
                    plsc.store_scatter(stage, [dtv[g], dsv, blv], vs[g])
                vs = nxt
            pltpu.sync_copy(stage.at[:, :, pl.ds(0, ch)],
                            o_hbm.at[c, slice(None), wid])

        def wait(b):
            pltpu.make_async_copy(w_hbm.at[idx_v.at[pl.ds(0, ch)]],
                                  rows[b], sems[b]).wait()

        start(0, 0)
        start(1, 1)
        start(2, 2)

        def body(o, carry):
            c0 = o * 4
            for b in range(4):
                c = c0 + b
                nb = (b + 3) % 4

                @pl.when(c + 3 < n_ch)
                def _():
                    start(c + 3, nb)

                wait(b)
                consume(c, b)
            return carry

        lax.fori_loop(0, n_ch // 4, body, 0)
        for tb, tc in enumerate(range((n_ch // 4) * 4, n_ch)):
            wait(tb)
            consume(tc, tb)

    return k(table, idx)


def kernel(input, A, S):
    n, mn, qn = S.shape
    d = n * qn
    w2 = _build_weight(A, S, n, qn)
    table = w2.reshape(n * mn, d)
    bsz, L = input.shape
    m = input.astype(jnp.int32).reshape(bsz // 128, 128, L)
    m = m.transpose(0, 2, 1).reshape(-1)
    j = m // mn
    p = m - j * mn
    idx = (j & 1) * (2 * mn) + 2 * p + (j >> 1)
    out5 = _gather_rows(table, idx, L, d)
    return out5.transpose(2, 4, 0, 1, 3).reshape(bsz, L, d)


# speedup vs baseline: 1.1255x; 1.1255x over previous
import functools

import jax
import jax.numpy as jnp
from jax import lax
from jax.experimental import pallas as pl
from jax.experimental.pallas import tpu as pltpu
from jax.experimental.pallas import tpu_sc as plsc


def _build_body(a_ref, s_ref, o_ref, *, n, qn):
    j2 = pl.program_id(0)
    d = n * qn
    qa = lax.broadcasted_iota(jnp.int32, (qn, 2 * d), 0)
    bi = lax.broadcasted_iota(jnp.int32, (qn, 2 * d), 1)
    cb = bi % d
    qb = cb % qn
    kb = cb // qn
    half = bi // d
    eyem = (qa == qb).astype(jnp.float32)
    acc = None
    for i in range(n):
        coef = jnp.zeros((qn, 2 * d), jnp.float32)
        for k in range(n):
            msk = (kb == k)
            coef = coef + jnp.where(msk & (half == 0), a_ref[i, j2, k], 0.0)
            coef = coef + jnp.where(msk & (half == 1), a_ref[i, j2 + 2, k], 0.0)
        t = jnp.dot(s_ref[i], eyem * coef, preferred_element_type=jnp.float32)
        acc = t if acc is None else acc + t
    o_ref[...] = acc


def _build_weight(A, S, n, qn):
    mn = S.shape[1]
    d = n * qn
    pb = 5000
    npb = mn // pb
    return pl.pallas_call(
        functools.partial(_build_body, n=n, qn=qn),
        grid=(2, npb),
        in_specs=[
            pl.BlockSpec(memory_space=pltpu.SMEM),
            pl.BlockSpec((n, pb, qn), lambda j2, p: (0, p, 0)),
        ],
        out_specs=pl.BlockSpec((pb, 2 * d), lambda j2, p: (j2 * npb + p, 0)),
        out_shape=jax.ShapeDtypeStruct((2 * mn, 2 * d), jnp.float32),
    )(A, S)


def _gather_rows(table, idx, L, d):
    info = plsc.get_sparse_core_info()
    nw = info.num_cores * info.num_subcores
    t = idx.shape[0]
    per_w = t // nw
    ch = 128
    n_ch = per_w // ch
    ndt = d // 8
    mesh = plsc.VectorSubcoreMesh(core_axis_name="c", subcore_axis_name="s")

    @functools.partial(
        pl.kernel,
        mesh=mesh,
        out_type=jax.ShapeDtypeStruct((L, ndt, nw, 8, ch), jnp.float32),
        scratch_types=[
            pltpu.VMEM((per_w,), jnp.int32),
            pltpu.VMEM((ch, d), jnp.float32),
            pltpu.VMEM((ch, d), jnp.float32),
            pltpu.VMEM((ndt, 8, ch + 1), jnp.float32),
            pltpu.VMEM((ndt, 8, ch + 1), jnp.float32),
            pltpu.SemaphoreType.DMA,
            pltpu.SemaphoreType.DMA,
            pltpu.SemaphoreType.DMA,
            pltpu.SemaphoreType.DMA,
        ],
        compiler_params=pltpu.CompilerParams(
            use_tc_tiling_on_sc=False, needs_layout_passes=False),
    )
    def k(w_hbm, i_hbm, o_hbm, idx_v, rows0, rows1, stage0, stage1,
          sem0, sem1, semo0, semo1):
        wid = lax.axis_index("s") * info.num_cores + lax.axis_index("c")
        base = wid * per_w
        pltpu.sync_copy(i_hbm.at[pl.ds(base, per_w)], idx_v)

        rows = (rows0, rows1)
        sems = (sem0, sem1)
        stages = (stage0, stage1)
        semos = (semo0, semo1)
        lane = lax.broadcasted_iota(jnp.int32, (16,), 0)
        dtv = [g * 2 + lane // 8 for g in range(d // 16)]
        dsv = lane % 8

        def start(c, b):
            return pltpu.async_copy(
                w_hbm.at[idx_v.at[pl.ds(c * ch, ch)]], rows[b], sems[b])

        def consume(c, b):
            stage = stages[b]

            @pl.when(c >= 2)
            def _():
                pltpu.make_async_copy(stage.at[:, :, pl.ds(0, ch)],
                                      o_hbm.at[c, slice(None), wid],
                                      semos[b]).wait()

            ng = d // 16
            vs = [rows[b][0, pl.ds(g * 16, 16)] for g in range(ng)]
            for bl in range(ch):
                nxt = ([rows[b][bl + 1, pl.ds(g * 16, 16)] for g in range(ng)]
                       if bl + 1 < ch else None)
                blv = jnp.full((16,), bl, jnp.int32)
                for g in range(ng):
                    plsc.store_scatter(stage, [dtv[g], dsv, blv], vs[g])
                vs = nxt
            pltpu.async_copy(stage.at[:, :, pl.ds(0, ch)],
                             o_hbm.at[c, slice(None), wid], semos[b])

        start(0, 0)

        def body(o, carry):
            c0 = o * 2
            start(c0 + 1, 1)
            pltpu.make_async_copy(w_hbm.at[idx_v.at[pl.ds(0, ch)]],
                                  rows[0], sems[0]).wait()
            consume(c0, 0)

            @pl.when(c0 + 2 < n_ch)
            def _():
                start(c0 + 2, 0)

            pltpu.make_async_copy(w_hbm.at[idx_v.at[pl.ds(0, ch)]],
                                  rows[1], sems[1]).wait()
            consume(c0 + 1, 1)
            return carry

        lax.fori_loop(0, n_ch // 2, body, 0)
        for b in range(2):
            pltpu.make_async_copy(
                stages[b].at[:, :, pl.ds(0, ch)],
                o_hbm.at[n_ch - 2 + b, slice(None), wid], semos[b]).wait()

    return k(table, idx)


def kernel(input, A, S):
    n, mn, qn = S.shape
    d = n * qn
    w2 = _build_weight(A, S, n, qn)
    table = w2.reshape(n * mn, d)
    bsz, L = input.shape
    m = input.astype(jnp.int32).reshape(bsz // 128, 128, L)
    m = m.transpose(0, 2, 1).reshape(-1)
    j = m // mn
    p = m - j * mn
    idx = (j & 1) * (2 * mn) + 2 * p + (j >> 1)
    out5 = _gather_rows(table, idx, L, d)
    return out5.transpose(2, 4, 0, 1, 3).reshape(bsz, L, d)
